# baseline (speedup 1.0000x reference)
import functools

import jax
import jax.numpy as jnp
from jax import lax
from jax.experimental import pallas as pl
from jax.experimental.pallas import tpu as pltpu
from jax.experimental.pallas import tpu_sc as plsc

B, H, GY, GX = 16, 4, 64, 64
NUM_OBJ = 5000
NUM_ASSIGN = 20000

NC, NS, L = 2, 16, 16
NW = NC * NS
CHUNK = 640
NV = CHUNK // L


def _sc_body(pred_hbm, hr_hbm, sc_hbm, obj_hbm, img_hbm, head_hbm, gy_hbm,
             gx_hbm, out_hbm,
             hr_tab, sc_tab,
             obj_v, img_v, head_v, gy_v, gx_v,
             ip_v, p_v,
             acc_v, sem_idx, sem_tab):
    cid = lax.axis_index("c")
    sid = lax.axis_index("s")
    wid = sid * NC + cid
    own = wid * CHUNK
    base = jnp.minimum(own, NUM_ASSIGN - CHUNK)

    ct0 = pltpu.async_copy(hr_hbm, hr_tab, sem_tab)
    ct1 = pltpu.async_copy(sc_hbm, sc_tab, sem_tab)
    ci0 = pltpu.async_copy(obj_hbm.at[pl.ds(base, CHUNK)], obj_v, sem_idx)
    ci1 = pltpu.async_copy(img_hbm.at[pl.ds(base, CHUNK)], img_v, sem_idx)
    ci2 = pltpu.async_copy(head_hbm.at[pl.ds(base, CHUNK)], head_v, sem_idx)
    ci3 = pltpu.async_copy(gy_hbm.at[pl.ds(base, CHUNK)], gy_v, sem_idx)
    ci4 = pltpu.async_copy(gx_hbm.at[pl.ds(base, CHUNK)], gx_v, sem_idx)
    ci0.wait()
    ci1.wait()
    ci2.wait()
    ci3.wait()
    ci4.wait()

    @plsc.parallel_loop(0, NV, 1, unroll=4)
    def idx_body(i):
        sl = pl.ds(i * L, L)
        flat = ((img_v[sl] * H + head_v[sl]) * 2) * (GY * GX) \
            + gy_v[sl] * GX + gx_v[sl]
        ip_v[sl] = flat
        ip_v[pl.ds(CHUNK + i * L, L)] = flat + GY * GX

    cp = pltpu.async_copy(pred_hbm.at[ip_v], p_v, sem_tab)
    ct0.wait()
    ct1.wait()
    cp.wait()

    iota = lax.iota(jnp.int32, L)

    @plsc.parallel_loop(0, NV, 1, unroll=8,
                        carry=jnp.zeros((L,), jnp.float32))
    def red_body(i, acc):
        sl = pl.ds(i * L, L)
        obj = obj_v[sl]
        hr = plsc.load_gather(hr_tab, [obj])
        t0 = plsc.load_gather(sc_tab, [obj])
        t1 = plsc.load_gather(sc_tab, [obj + NUM_OBJ])
        pos = base + i * L + iota
        m = (hr != 0) & (pos >= own)
        d0 = t0 - p_v[sl]
        d1 = t1 - p_v[pl.ds(CHUNK + i * L, L)]
        return acc + jnp.where(m, d0 * d0 + d1 * d1, 0.0)

    acc_v[:] = red_body
    pltpu.sync_copy(acc_v, out_hbm.at[wid])


@jax.jit
def _sc_loss(pred_flat, has_rotation, sc_flat, obj, img, head, gy, gx):
    mesh = plsc.VectorSubcoreMesh(core_axis_name="c", subcore_axis_name="s")
    run = functools.partial(
        pl.kernel,
        mesh=mesh,
        compiler_params=pltpu.CompilerParams(needs_layout_passes=False,
                                             skip_device_barrier=True),
        out_type=jax.ShapeDtypeStruct((NW, L), jnp.float32),
        scratch_types=[
            pltpu.VMEM((NUM_OBJ,), jnp.int32),
            pltpu.VMEM((2 * NUM_OBJ,), jnp.float32),
            pltpu.VMEM((CHUNK,), jnp.int32),
            pltpu.VMEM((CHUNK,), jnp.int32),
            pltpu.VMEM((CHUNK,), jnp.int32),
            pltpu.VMEM((CHUNK,), jnp.int32),
            pltpu.VMEM((CHUNK,), jnp.int32),
            pltpu.VMEM((2 * CHUNK,), jnp.int32),
            pltpu.VMEM((2 * CHUNK,), jnp.float32),
            pltpu.VMEM((L,), jnp.float32),
            pltpu.SemaphoreType.DMA,
            pltpu.SemaphoreType.DMA,
        ],
    )(_sc_body)
    out = run(pred_flat, has_rotation, sc_flat, obj, img, head, gy, gx)
    return jnp.sum(out)


def kernel(post_activation_sincos, has_rotation, sincos, object_idxs,
           img_idxs, head_idxs, grid_y_idxs, grid_x_idxs):
    return _sc_loss(post_activation_sincos.reshape(-1), has_rotation,
                    sincos.T.reshape(-1), object_idxs, img_idxs, head_idxs,
                    grid_y_idxs, grid_x_idxs)


# speedup vs baseline: 1.0058x; 1.0058x over previous
import functools

import jax
import jax.numpy as jnp
from jax import lax
from jax.experimental import pallas as pl
from jax.experimental.pallas import tpu as pltpu
from jax.experimental.pallas import tpu_sc as plsc

B, H, GY, GX = 16, 4, 64, 64
NUM_OBJ = 5000
NUM_ASSIGN = 20000

NC, NS, L = 2, 16, 16
NW = NC * NS
CHUNK = 640
NV = CHUNK // L


def _sc_body(pred_hbm, hr_hbm, sc_hbm, obj_hbm, img_hbm, head_hbm, gy_hbm,
             gx_hbm, out_hbm,
             hr_tab, sc_tab,
             obj_v, img_v, head_v, gy_v, gx_v,
             ip_v, p_v,
             acc_v, sem_idx, sem_tab):
    cid = lax.axis_index("c")
    sid = lax.axis_index("s")
    wid = sid * NC + cid
    own = wid * CHUNK
    base = jnp.minimum(own, NUM_ASSIGN - CHUNK)

    ct0 = pltpu.async_copy(hr_hbm, hr_tab, sem_tab)
    ct1 = pltpu.async_copy(sc_hbm, sc_tab, sem_tab)
    ci0 = pltpu.async_copy(obj_hbm.at[pl.ds(base, CHUNK)], obj_v, sem_idx)
    ci1 = pltpu.async_copy(img_hbm.at[pl.ds(base, CHUNK)], img_v, sem_idx)
    ci2 = pltpu.async_copy(head_hbm.at[pl.ds(base, CHUNK)], head_v, sem_idx)
    ci3 = pltpu.async_copy(gy_hbm.at[pl.ds(base, CHUNK)], gy_v, sem_idx)
    ci4 = pltpu.async_copy(gx_hbm.at[pl.ds(base, CHUNK)], gx_v, sem_idx)
    ci0.wait()
    ci1.wait()
    ci2.wait()
    ci3.wait()
    ci4.wait()

    @plsc.parallel_loop(0, NV, 1, unroll=4)
    def idx_body(i):
        sl = pl.ds(i * L, L)
        flat = ((img_v[sl] * H + head_v[sl]) * 2) * (GY * GX) \
            + gy_v[sl] * GX + gx_v[sl]
        ip_v[sl] = flat
        ip_v[pl.ds(CHUNK + i * L, L)] = flat + GY * GX

    cp = pltpu.async_copy(pred_hbm.at[ip_v], p_v, sem_tab)
    ct0.wait()
    ct1.wait()
    cp.wait()

    iota = lax.iota(jnp.int32, L)

    @plsc.parallel_loop(0, NV, 1, unroll=4,
                        carry=jnp.zeros((L,), jnp.float32))
    def red_body(i, acc):
        sl = pl.ds(i * L, L)
        obj = obj_v[sl]
        hr = plsc.load_gather(hr_tab, [obj])
        t0 = plsc.load_gather(sc_tab, [obj])
        t1 = plsc.load_gather(sc_tab, [obj + NUM_OBJ])
        pos = base + i * L + iota
        m = (hr != 0) & (pos >= own)
        d0 = t0 - p_v[sl]
        d1 = t1 - p_v[pl.ds(CHUNK + i * L, L)]
        return acc + jnp.where(m, d0 * d0 + d1 * d1, 0.0)

    acc_v[:] = red_body
    pltpu.sync_copy(acc_v, out_hbm.at[wid])


@jax.jit
def _sc_loss(pred_flat, has_rotation, sc_flat, obj, img, head, gy, gx):
    mesh = plsc.VectorSubcoreMesh(core_axis_name="c", subcore_axis_name="s")
    run = functools.partial(
        pl.kernel,
        mesh=mesh,
        compiler_params=pltpu.CompilerParams(needs_layout_passes=False,
                                             skip_device_barrier=True),
        out_type=jax.ShapeDtypeStruct((NW, L), jnp.float32),
        scratch_types=[
            pltpu.VMEM((NUM_OBJ,), jnp.int32),
            pltpu.VMEM((2 * NUM_OBJ,), jnp.float32),
            pltpu.VMEM((CHUNK,), jnp.int32),
            pltpu.VMEM((CHUNK,), jnp.int32),
            pltpu.VMEM((CHUNK,), jnp.int32),
            pltpu.VMEM((CHUNK,), jnp.int32),
            pltpu.VMEM((CHUNK,), jnp.int32),
            pltpu.VMEM((2 * CHUNK,), jnp.int32),
            pltpu.VMEM((2 * CHUNK,), jnp.float32),
            pltpu.VMEM((L,), jnp.float32),
            pltpu.SemaphoreType.DMA,
            pltpu.SemaphoreType.DMA,
        ],
    )(_sc_body)
    out = run(pred_flat, has_rotation, sc_flat, obj, img, head, gy, gx)
    return jnp.sum(out)


def kernel(post_activation_sincos, has_rotation, sincos, object_idxs,
           img_idxs, head_idxs, grid_y_idxs, grid_x_idxs):
    return _sc_loss(post_activation_sincos.reshape(-1), has_rotation,
                    sincos.T.reshape(-1), object_idxs, img_idxs, head_idxs,
                    grid_y_idxs, grid_x_idxs)


# speedup vs baseline: 1.0411x; 1.0351x over previous
import functools

import jax
import jax.numpy as jnp
from jax import lax
from jax.experimental import pallas as pl
from jax.experimental.pallas import tpu as pltpu
from jax.experimental.pallas import tpu_sc as plsc

B, H, GY, GX = 16, 4, 64, 64
NUM_OBJ = 5000
NUM_ASSIGN = 20000

NC, NS, L = 2, 16, 16
NW = NC * NS
CHUNK = 640
NV = CHUNK // L


def _sc_body(pred_hbm, hr_hbm, sc_hbm, obj_hbm, img_hbm, head_hbm, gy_hbm,
             gx_hbm, out_hbm,
             hr_tab, sc_tab,
             obj_v, img_v, head_v, gy_v, gx_v,
             ip_v, p_v,
             acc_v, sem_idx, sem_tab):
    cid = lax.axis_index("c")
    sid = lax.axis_index("s")
    wid = sid * NC + cid
    own = wid * CHUNK
    base = jnp.minimum(own, NUM_ASSIGN - CHUNK)

    ct0 = pltpu.async_copy(hr_hbm, hr_tab, sem_tab)
    ct1 = pltpu.async_copy(
        sc_hbm.at[pl.ds(B * H * 2 * GY * GX, 2 * NUM_OBJ)], sc_tab, sem_tab)
    ci0 = pltpu.async_copy(obj_hbm.at[pl.ds(base, CHUNK)], obj_v, sem_idx)
    ci1 = pltpu.async_copy(img_hbm.at[pl.ds(base, CHUNK)], img_v, sem_idx)
    ci2 = pltpu.async_copy(head_hbm.at[pl.ds(base, CHUNK)], head_v, sem_idx)
    ci3 = pltpu.async_copy(gy_hbm.at[pl.ds(base, CHUNK)], gy_v, sem_idx)
    ci4 = pltpu.async_copy(gx_hbm.at[pl.ds(base, CHUNK)], gx_v, sem_idx)
    ci0.wait()
    ci1.wait()
    ci2.wait()
    ci3.wait()
    ci4.wait()

    @plsc.parallel_loop(0, NV, 1, unroll=4)
    def idx_body(i):
        sl = pl.ds(i * L, L)
        flat = ((img_v[sl] * H + head_v[sl]) * 2) * (GY * GX) \
            + gy_v[sl] * GX + gx_v[sl]
        ip_v[sl] = flat
        ip_v[pl.ds(CHUNK + i * L, L)] = flat + GY * GX

    cp = pltpu.async_copy(pred_hbm.at[ip_v], p_v, sem_tab)
    ct0.wait()
    ct1.wait()
    cp.wait()

    iota = lax.iota(jnp.int32, L)

    @plsc.parallel_loop(0, NV, 1, unroll=4,
                        carry=jnp.zeros((L,), jnp.float32))
    def red_body(i, acc):
        sl = pl.ds(i * L, L)
        obj = obj_v[sl]
        hr = plsc.load_gather(hr_tab, [obj])
        t0 = plsc.load_gather(sc_tab, [obj])
        t1 = plsc.load_gather(sc_tab, [obj + NUM_OBJ])
        pos = base + i * L + iota
        m = (hr != 0) & (pos >= own)
        d0 = t0 - p_v[sl]
        d1 = t1 - p_v[pl.ds(CHUNK + i * L, L)]
        return acc + jnp.where(m, d0 * d0 + d1 * d1, 0.0)

    acc_v[:] = red_body
    pltpu.sync_copy(acc_v, out_hbm.at[wid])


@jax.jit
def _sc_loss(pred_flat, has_rotation, sc_flat, obj, img, head, gy, gx):
    mesh = plsc.VectorSubcoreMesh(core_axis_name="c", subcore_axis_name="s")
    run = functools.partial(
        pl.kernel,
        mesh=mesh,
        compiler_params=pltpu.CompilerParams(needs_layout_passes=False,
                                             skip_device_barrier=True),
        out_type=jax.ShapeDtypeStruct((NW, L), jnp.float32),
        scratch_types=[
            pltpu.VMEM((NUM_OBJ,), jnp.int32),
            pltpu.VMEM((2 * NUM_OBJ,), jnp.float32),
            pltpu.VMEM((CHUNK,), jnp.int32),
            pltpu.VMEM((CHUNK,), jnp.int32),
            pltpu.VMEM((CHUNK,), jnp.int32),
            pltpu.VMEM((CHUNK,), jnp.int32),
            pltpu.VMEM((CHUNK,), jnp.int32),
            pltpu.VMEM((2 * CHUNK,), jnp.int32),
            pltpu.VMEM((2 * CHUNK,), jnp.float32),
            pltpu.VMEM((L,), jnp.float32),
            pltpu.SemaphoreType.DMA,
            pltpu.SemaphoreType.DMA,
        ],
    )(_sc_body)
    out = run(pred_flat, has_rotation, sc_flat, obj, img, head, gy, gx)
    return jnp.sum(out)


def kernel(post_activation_sincos, has_rotation, sincos, object_idxs,
           img_idxs, head_idxs, grid_y_idxs, grid_x_idxs):
    table = jnp.concatenate(
        [post_activation_sincos.reshape(-1), sincos.T.reshape(-1)])
    return _sc_loss(table, has_rotation, table, object_idxs, img_idxs,
                    head_idxs, grid_y_idxs, grid_x_idxs)


# speedup vs baseline: 1.0414x; 1.0003x over previous
import functools

import jax
import jax.numpy as jnp
from jax import lax
from jax.experimental import pallas as pl
from jax.experimental.pallas import tpu as pltpu
from jax.experimental.pallas import tpu_sc as plsc

B, H, GY, GX = 16, 4, 64, 64
NUM_OBJ = 5000
NUM_ASSIGN = 20000

NC, NS, L = 2, 16, 16
NW = NC * NS
CHUNK = 640
NV = CHUNK // L


def _sc_body(pred_hbm, hr_hbm, sc_hbm, obj_hbm, img_hbm, head_hbm, gy_hbm,
             gx_hbm, out_hbm,
             hr_tab, sc_tab,
             obj_v, img_v, head_v, gy_v, gx_v,
             ip_v, p_v,
             acc_v, sem_idx, sem_tab):
    cid = lax.axis_index("c")
    sid = lax.axis_index("s")
    wid = sid * NC + cid
    own = wid * CHUNK
    base = jnp.minimum(own, NUM_ASSIGN - CHUNK)

    ct0 = pltpu.async_copy(hr_hbm, hr_tab, sem_tab)
    ct1 = pltpu.async_copy(
        sc_hbm.at[pl.ds(B * H * 2 * GY * GX, 2 * NUM_OBJ)], sc_tab, sem_tab)
    ci0 = pltpu.async_copy(obj_hbm.at[pl.ds(base, CHUNK)], obj_v, sem_idx)
    ci1 = pltpu.async_copy(img_hbm.at[pl.ds(base, CHUNK)], img_v, sem_idx)
    ci2 = pltpu.async_copy(head_hbm.at[pl.ds(base, CHUNK)], head_v, sem_idx)
    ci3 = pltpu.async_copy(gy_hbm.at[pl.ds(base, CHUNK)], gy_v, sem_idx)
    ci4 = pltpu.async_copy(gx_hbm.at[pl.ds(base, CHUNK)], gx_v, sem_idx)
    ci0.wait()
    ci1.wait()
    ci2.wait()
    ci3.wait()
    ci4.wait()

    @plsc.parallel_loop(0, NV, 1, unroll=2)
    def idx_body(i):
        sl = pl.ds(i * L, L)
        flat = ((img_v[sl] * H + head_v[sl]) * 2) * (GY * GX) \
            + gy_v[sl] * GX + gx_v[sl]
        ip_v[sl] = flat
        ip_v[pl.ds(CHUNK + i * L, L)] = flat + GY * GX

    cp = pltpu.async_copy(pred_hbm.at[ip_v], p_v, sem_tab)
    ct0.wait()
    ct1.wait()
    cp.wait()

    iota = lax.iota(jnp.int32, L)

    @plsc.parallel_loop(0, NV, 1, unroll=2,
                        carry=jnp.zeros((L,), jnp.float32))
    def red_body(i, acc):
        sl = pl.ds(i * L, L)
        obj = obj_v[sl]
        hr = plsc.load_gather(hr_tab, [obj])
        t0 = plsc.load_gather(sc_tab, [obj])
        t1 = plsc.load_gather(sc_tab, [obj + NUM_OBJ])
        pos = base + i * L + iota
        m = (hr != 0) & (pos >= own)
        d0 = t0 - p_v[sl]
        d1 = t1 - p_v[pl.ds(CHUNK + i * L, L)]
        return acc + jnp.where(m, d0 * d0 + d1 * d1, 0.0)

    acc_v[:] = red_body
    pltpu.sync_copy(acc_v, out_hbm.at[wid])


@jax.jit
def _sc_loss(pred_flat, has_rotation, sc_flat, obj, img, head, gy, gx):
    mesh = plsc.VectorSubcoreMesh(core_axis_name="c", subcore_axis_name="s")
    run = functools.partial(
        pl.kernel,
        mesh=mesh,
        compiler_params=pltpu.CompilerParams(needs_layout_passes=False,
                                             skip_device_barrier=True),
        out_type=jax.ShapeDtypeStruct((NW, L), jnp.float32),
        scratch_types=[
            pltpu.VMEM((NUM_OBJ,), jnp.int32),
            pltpu.VMEM((2 * NUM_OBJ,), jnp.float32),
            pltpu.VMEM((CHUNK,), jnp.int32),
            pltpu.VMEM((CHUNK,), jnp.int32),
            pltpu.VMEM((CHUNK,), jnp.int32),
            pltpu.VMEM((CHUNK,), jnp.int32),
            pltpu.VMEM((CHUNK,), jnp.int32),
            pltpu.VMEM((2 * CHUNK,), jnp.int32),
            pltpu.VMEM((2 * CHUNK,), jnp.float32),
            pltpu.VMEM((L,), jnp.float32),
            pltpu.SemaphoreType.DMA,
            pltpu.SemaphoreType.DMA,
        ],
    )(_sc_body)
    out = run(pred_flat, has_rotation, sc_flat, obj, img, head, gy, gx)
    return jnp.sum(out)


def kernel(post_activation_sincos, has_rotation, sincos, object_idxs,
           img_idxs, head_idxs, grid_y_idxs, grid_x_idxs):
    table = jnp.concatenate(
        [post_activation_sincos.reshape(-1), sincos.T.reshape(-1)])
    return _sc_loss(table, has_rotation, table, object_idxs, img_idxs,
                    head_idxs, grid_y_idxs, grid_x_idxs)


# speedup vs baseline: 1.0442x; 1.0027x over previous
import functools

import jax
import jax.numpy as jnp
from jax import lax
from jax.experimental import pallas as pl
from jax.experimental.pallas import tpu as pltpu
from jax.experimental.pallas import tpu_sc as plsc

B, H, GY, GX = 16, 4, 64, 64
NUM_OBJ = 5000
NUM_ASSIGN = 20000

NC, NS, L = 2, 16, 16
NW = NC * NS
CHUNK = 640
NV = CHUNK // L


def _sc_body(pred_hbm, hr_hbm, sc_hbm, obj_hbm, img_hbm, head_hbm, gy_hbm,
             gx_hbm, out_hbm,
             hr_tab, sc_tab,
             obj_v, img_v, head_v, gy_v, gx_v,
             ip_v, p_v,
             acc_v, sem_idx, sem_tab):
    cid = lax.axis_index("c")
    sid = lax.axis_index("s")
    wid = sid * NC + cid
    own = wid * CHUNK
    base = jnp.minimum(own, NUM_ASSIGN - CHUNK)

    ct0 = pltpu.async_copy(hr_hbm, hr_tab, sem_tab)
    ct1 = pltpu.async_copy(
        sc_hbm.at[pl.ds(B * H * 2 * GY * GX, 2 * NUM_OBJ)], sc_tab, sem_tab)
    ci0 = pltpu.async_copy(obj_hbm.at[pl.ds(base, CHUNK)], obj_v, sem_idx)
    ci1 = pltpu.async_copy(img_hbm.at[pl.ds(base, CHUNK)], img_v, sem_idx)
    ci2 = pltpu.async_copy(head_hbm.at[pl.ds(base, CHUNK)], head_v, sem_idx)
    ci3 = pltpu.async_copy(gy_hbm.at[pl.ds(base, CHUNK)], gy_v, sem_idx)
    ci4 = pltpu.async_copy(gx_hbm.at[pl.ds(base, CHUNK)], gx_v, sem_idx)
    ci0.wait()
    ci1.wait()
    ci2.wait()
    ci3.wait()
    ci4.wait()

    @plsc.parallel_loop(0, NV, 1, unroll=2)
    def idx_body(i):
        sl = pl.ds(i * L, L)
        flat = ((img_v[sl] * H + head_v[sl]) * 2) * (GY * GX) \
            + gy_v[sl] * GX + gx_v[sl]
        ip_v[sl] = flat
        ip_v[pl.ds(CHUNK + i * L, L)] = flat + GY * GX

    cp = pltpu.async_copy(pred_hbm.at[ip_v], p_v, sem_tab)
    ct0.wait()
    ct1.wait()
    cp.wait()

    iota = lax.iota(jnp.int32, L)

    @plsc.parallel_loop(0, NV, 1, unroll=2,
                        carry=jnp.zeros((L,), jnp.float32))
    def red_body(i, acc):
        sl = pl.ds(i * L, L)
        obj = obj_v[sl]
        hr = plsc.load_gather(hr_tab, [obj])
        t0 = plsc.load_gather(sc_tab, [obj])
        t1 = plsc.load_gather(sc_tab, [obj + NUM_OBJ])
        pos = base + i * L + iota
        m = (hr != 0) & (pos >= own)
        d0 = t0 - p_v[sl]
        d1 = t1 - p_v[pl.ds(CHUNK + i * L, L)]
        return acc + jnp.where(m, d0 * d0 + d1 * d1, 0.0)

    acc_v[:] = red_body
    pltpu.sync_copy(acc_v, out_hbm.at[wid])


@jax.jit
def _sc_loss(pred_flat, has_rotation, sc_flat, obj, img, head, gy, gx):
    mesh = plsc.VectorSubcoreMesh(core_axis_name="c", subcore_axis_name="s")
    run = functools.partial(
        pl.kernel,
        mesh=mesh,
        compiler_params=pltpu.CompilerParams(needs_layout_passes=False),
        out_type=jax.ShapeDtypeStruct((NW, L), jnp.float32),
        scratch_types=[
            pltpu.VMEM((NUM_OBJ,), jnp.int32),
            pltpu.VMEM((2 * NUM_OBJ,), jnp.float32),
            pltpu.VMEM((CHUNK,), jnp.int32),
            pltpu.VMEM((CHUNK,), jnp.int32),
            pltpu.VMEM((CHUNK,), jnp.int32),
            pltpu.VMEM((CHUNK,), jnp.int32),
            pltpu.VMEM((CHUNK,), jnp.int32),
            pltpu.VMEM((2 * CHUNK,), jnp.int32),
            pltpu.VMEM((2 * CHUNK,), jnp.float32),
            pltpu.VMEM((L,), jnp.float32),
            pltpu.SemaphoreType.DMA,
            pltpu.SemaphoreType.DMA,
        ],
    )(_sc_body)
    out = run(pred_flat, has_rotation, sc_flat, obj, img, head, gy, gx)
    return jnp.sum(out)


def kernel(post_activation_sincos, has_rotation, sincos, object_idxs,
           img_idxs, head_idxs, grid_y_idxs, grid_x_idxs):
    table = jnp.concatenate(
        [post_activation_sincos.reshape(-1), sincos.T.reshape(-1)])
    return _sc_loss(table, has_rotation, table, object_idxs, img_idxs,
                    head_idxs, grid_y_idxs, grid_x_idxs)
